# baseline (speedup 1.0000x reference)
import functools

import jax
import jax.numpy as jnp
from jax.experimental import pallas as pl

B, N, D, H, O, TOPK = 1024, 50, 512, 512, 512, 16
BBLK = 64
R = BBLK * N


def _bf16(x):
    return x.astype(jnp.bfloat16)


def _dot(a, b, dims):
    return jax.lax.dot_general(a, b, (dims, ((), ())),
                               preferred_element_type=jnp.float32)


def _fused_kernel(c_ref, e_ref, wq_ref, bq_ref, wk_ref, bk_ref, wv_ref,
                  bv_ref, wo_ref, bo_ref, o_ref):
    row = jax.lax.broadcasted_iota(jnp.int32, (R, BBLK), 0)
    col = jax.lax.broadcasted_iota(jnp.int32, (R, BBLK), 1)
    in_group = (row >= col * N) & (row < (col + 1) * N)

    e16 = _bf16(e_ref[...])
    q = _dot(_bf16(c_ref[...]), _bf16(wq_ref[...]), ((1,), (1,)))
    q16 = _bf16(q + bq_ref[...])
    k16 = _bf16(_dot(e16, _bf16(wk_ref[...]), ((1,), (1,))) + bk_ref[...])
    ell = _dot(k16, q16, ((1,), (1,)))
    lflat = jnp.sum(jnp.where(in_group, ell, 0.0), axis=1,
                    keepdims=True) * (H ** -0.5)
    logits = lflat.reshape(BBLK, N)
    m = jnp.max(logits, axis=-1, keepdims=True)
    p = jnp.exp(logits - m)
    s = p / jnp.sum(p, axis=-1, keepdims=True)
    lane = jax.lax.broadcasted_iota(jnp.int32, (1, N), 1)
    rank_cols = []
    for n in range(N):
        sn = s[:, n:n + 1]
        beats = (s > sn) | ((s == sn) & (lane < n))
        rank_cols.append(jnp.sum(beats.astype(jnp.int32), axis=-1,
                                 keepdims=True))
    rank = jnp.concatenate(rank_cols, axis=1)
    w = jnp.where(rank < TOPK, s, 0.0)
    s_tot = jnp.sum(w, axis=-1, keepdims=True)
    t = _dot(_bf16(in_group.astype(jnp.float32)), _bf16(w), ((1,), (0,)))
    b_of_r = jnp.sum(jnp.where(in_group, col, 0), axis=1, keepdims=True)
    n_of_r = row[:, 0:1] - N * b_of_r
    lane_n = jax.lax.broadcasted_iota(jnp.int32, (R, N), 1)
    w_col = jnp.sum(jnp.where(lane_n == n_of_r, t, 0.0), axis=1,
                    keepdims=True)
    wcb = jnp.where(in_group, w_col, 0.0)
    e_w = _dot(_bf16(wcb), e16, ((0,), (0,)))
    ws = _dot(_bf16(e_w), _bf16(wv_ref[...]), ((1,), (1,)))
    ws = ws + s_tot * bv_ref[...]
    o_ref[...] = _dot(_bf16(ws), _bf16(wo_ref[...]), ((1,), (1,))) \
        + bo_ref[...]


@functools.partial(jax.jit, static_argnames=("interpret",))
def _run(class_embedding, entity_embeddings, Wq, bq, Wk, bk, Wv, bv, Wo, bo,
         interpret=False):
    grid = (B // BBLK,)
    full = lambda *shape: pl.BlockSpec(shape, lambda i: (0,) * len(shape))
    return pl.pallas_call(
        _fused_kernel,
        grid=grid,
        in_specs=[
            pl.BlockSpec((BBLK, D), lambda i: (i, 0)),
            pl.BlockSpec((R, D), lambda i: (i, 0)),
            full(H, D),
            full(1, H),
            full(H, D),
            full(1, H),
            full(H, D),
            full(1, H),
            full(O, H),
            full(1, O),
        ],
        out_specs=pl.BlockSpec((BBLK, O), lambda i: (i, 0)),
        out_shape=jax.ShapeDtypeStruct((B, O), jnp.float32),
        interpret=interpret,
    )(class_embedding, entity_embeddings.reshape(B * N, D), Wq,
      bq.reshape(1, H), Wk, bk.reshape(1, H), Wv, bv.reshape(1, H), Wo,
      bo.reshape(1, O))


def kernel(class_embedding, entity_embeddings, Wq, bq, Wk, bk, Wv, bv, Wo, bo):
    return _run(class_embedding, entity_embeddings, Wq, bq, Wk, bk, Wv, bv,
                Wo, bo)


# speedup vs baseline: 9.7634x; 9.7634x over previous
import functools

import jax
import jax.numpy as jnp
from jax.experimental import pallas as pl

B, N, D, H, O, TOPK = 1024, 50, 512, 512, 512, 16
BBLK = 64
R = BBLK * N


def _bf16(x):
    return x.astype(jnp.bfloat16)


def _dot(a, b, dims, precision=None):
    return jax.lax.dot_general(a, b, (dims, ((), ())),
                               preferred_element_type=jnp.float32,
                               precision=precision)


def _fused_kernel(c_ref, e_ref, wq_ref, bq_ref, wk_ref, bk_ref, wv_ref,
                  bv_ref, wo_ref, bo_ref, o_ref):
    row = jax.lax.broadcasted_iota(jnp.int32, (R, BBLK), 0)
    col = jax.lax.broadcasted_iota(jnp.int32, (R, BBLK), 1)
    in_group = (row >= col * N) & (row < (col + 1) * N)
    rql = jax.lax.broadcasted_iota(jnp.int32, (R, N), 0)
    nql = jax.lax.broadcasted_iota(jnp.int32, (R, N), 1)
    qsel = (rql - N * jax.lax.div(rql, N)) == nql
    qself = qsel.astype(jnp.float32)

    e16 = _bf16(e_ref[...])
    q = _dot(_bf16(c_ref[...]), _bf16(wq_ref[...]), ((1,), (1,)))
    q16 = _bf16(q + bq_ref[...])
    k16 = _bf16(_dot(e16, _bf16(wk_ref[...]), ((1,), (1,))) + bk_ref[...])
    ell = _dot(k16, q16, ((1,), (1,)))
    masked = jnp.where(in_group, ell, 0.0)
    logits = _dot(masked, qself, ((0,), (0,)),
                  precision=jax.lax.Precision.HIGHEST) * (H ** -0.5)
    m = jnp.max(logits, axis=-1, keepdims=True)
    p = jnp.exp(logits - m)
    s = p / jnp.sum(p, axis=-1, keepdims=True)
    lane = jax.lax.broadcasted_iota(jnp.int32, (1, N), 1)
    rank_cols = []
    for n in range(N):
        sn = s[:, n:n + 1]
        beats = (s > sn) | ((s == sn) & (lane < n))
        rank_cols.append(jnp.sum(beats.astype(jnp.int32), axis=-1,
                                 keepdims=True))
    rank = jnp.concatenate(rank_cols, axis=1)
    w = jnp.where(rank < TOPK, s, 0.0)
    s_tot = jnp.sum(w, axis=-1, keepdims=True)
    t = _dot(qself, w, ((1,), (1,)))
    wcb16 = _bf16(jnp.where(in_group, t, 0.0))
    e_w = _dot(wcb16, e16, ((0,), (0,)))
    ws = _dot(_bf16(e_w), _bf16(wv_ref[...]), ((1,), (1,)))
    ws = ws + s_tot * bv_ref[...]
    o_ref[...] = _dot(_bf16(ws), _bf16(wo_ref[...]), ((1,), (1,))) \
        + bo_ref[...]


@functools.partial(jax.jit, static_argnames=("interpret",))
def _run(class_embedding, entity_embeddings, Wq, bq, Wk, bk, Wv, bv, Wo, bo,
         interpret=False):
    grid = (B // BBLK,)
    full = lambda *shape: pl.BlockSpec(shape, lambda i: (0,) * len(shape))
    return pl.pallas_call(
        _fused_kernel,
        grid=grid,
        in_specs=[
            pl.BlockSpec((BBLK, D), lambda i: (i, 0)),
            pl.BlockSpec((R, D), lambda i: (i, 0)),
            full(H, D),
            full(1, H),
            full(H, D),
            full(1, H),
            full(H, D),
            full(1, H),
            full(O, H),
            full(1, O),
        ],
        out_specs=pl.BlockSpec((BBLK, O), lambda i: (i, 0)),
        out_shape=jax.ShapeDtypeStruct((B, O), jnp.float32),
        interpret=interpret,
    )(class_embedding, entity_embeddings.reshape(B * N, D), Wq,
      bq.reshape(1, H), Wk, bk.reshape(1, H), Wv, bv.reshape(1, H), Wo,
      bo.reshape(1, O))


def kernel(class_embedding, entity_embeddings, Wq, bq, Wk, bk, Wv, bv, Wo, bo):
    return _run(class_embedding, entity_embeddings, Wq, bq, Wk, bk, Wv, bv,
                Wo, bo)


# speedup vs baseline: 10.0285x; 1.0271x over previous
import functools

import jax
import jax.numpy as jnp
from jax.experimental import pallas as pl

B, N, D, H, O, TOPK = 1024, 50, 512, 512, 512, 16
BBLK = 64
R = BBLK * N


def _bf16(x):
    return x.astype(jnp.bfloat16)


def _dot(a, b, dims, precision=None):
    return jax.lax.dot_general(a, b, (dims, ((), ())),
                               preferred_element_type=jnp.float32,
                               precision=precision)


def _fused_kernel(c_ref, e_ref, wq_ref, bq_ref, wk_ref, bk_ref, wv_ref,
                  bv_ref, wo_ref, bo_ref, o_ref):
    row = jax.lax.broadcasted_iota(jnp.int32, (R, BBLK), 0)
    col = jax.lax.broadcasted_iota(jnp.int32, (R, BBLK), 1)
    in_group = (row >= col * N) & (row < (col + 1) * N)
    rql = jax.lax.broadcasted_iota(jnp.int32, (R, N), 0)
    nql = jax.lax.broadcasted_iota(jnp.int32, (R, N), 1)
    qsel = (rql - N * jax.lax.div(rql, N)) == nql
    qself = qsel.astype(jnp.float32)

    e16 = e_ref[...]
    q = _dot(c_ref[...], wq_ref[...], ((1,), (1,)))
    q16 = _bf16(q + bq_ref[...])
    k16 = _bf16(_dot(e16, wk_ref[...], ((1,), (1,))) + bk_ref[...])
    ell = _dot(k16, q16, ((1,), (1,)))
    masked = jnp.where(in_group, ell, 0.0)
    logits = _dot(masked, qself, ((0,), (0,)),
                  precision=jax.lax.Precision.HIGHEST) * (H ** -0.5)
    m = jnp.max(logits, axis=-1, keepdims=True)
    p = jnp.exp(logits - m)
    s = p / jnp.sum(p, axis=-1, keepdims=True)
    lane = jax.lax.broadcasted_iota(jnp.int32, (1, N), 1)
    rank_cols = []
    for n in range(N):
        sn = s[:, n:n + 1]
        beats = (s > sn) | ((s == sn) & (lane < n))
        rank_cols.append(jnp.sum(beats.astype(jnp.int32), axis=-1,
                                 keepdims=True))
    rank = jnp.concatenate(rank_cols, axis=1)
    w = jnp.where(rank < TOPK, s, 0.0)
    s_tot = jnp.sum(w, axis=-1, keepdims=True)
    t = _dot(qself, w, ((1,), (1,)))
    wcb16 = _bf16(jnp.where(in_group, t, 0.0))
    e_w = _dot(wcb16, e16, ((0,), (0,)))
    ws = _dot(_bf16(e_w), wv_ref[...], ((1,), (1,)))
    ws = ws + s_tot * bv_ref[...]
    o_ref[...] = _dot(_bf16(ws), wo_ref[...], ((1,), (1,))) \
        + bo_ref[...]


@functools.partial(jax.jit, static_argnames=("interpret",))
def _run(class_embedding, entity_embeddings, Wq, bq, Wk, bk, Wv, bv, Wo, bo,
         interpret=False):
    grid = (B // BBLK,)
    full = lambda *shape: pl.BlockSpec(shape, lambda i: (0,) * len(shape))
    return pl.pallas_call(
        _fused_kernel,
        grid=grid,
        in_specs=[
            pl.BlockSpec((BBLK, D), lambda i: (i, 0)),
            pl.BlockSpec((R, D), lambda i: (i, 0)),
            full(H, D),
            full(1, H),
            full(H, D),
            full(1, H),
            full(H, D),
            full(1, H),
            full(O, H),
            full(1, O),
        ],
        out_specs=pl.BlockSpec((BBLK, O), lambda i: (i, 0)),
        out_shape=jax.ShapeDtypeStruct((B, O), jnp.float32),
        interpret=interpret,
    )(class_embedding.astype(jnp.bfloat16),
      entity_embeddings.reshape(B * N, D).astype(jnp.bfloat16),
      Wq.astype(jnp.bfloat16), bq.reshape(1, H),
      Wk.astype(jnp.bfloat16), bk.reshape(1, H),
      Wv.astype(jnp.bfloat16), bv.reshape(1, H),
      Wo.astype(jnp.bfloat16), bo.reshape(1, O))


def kernel(class_embedding, entity_embeddings, Wq, bq, Wk, bk, Wv, bv, Wo, bo):
    return _run(class_embedding, entity_embeddings, Wq, bq, Wk, bk, Wv, bv,
                Wo, bo)
